# baseline (speedup 1.0000x reference)
import functools

import jax
import jax.numpy as jnp
from jax import lax
from jax.experimental import pallas as pl
from jax.experimental.pallas import tpu as pltpu
from jax.experimental.pallas import tpu_sc as plsc

NC = 2
NS = 16
L = 16
NW = NC * NS
CHUNK = 128
NBUF = 2
HALVES = 2


def _pieces(total):
    base = (total // NS) // 8 * 8
    main = [(m * CHUNK, min(CHUNK, base - m * CHUNK))
            for m in range(-(-base // CHUNK))]
    tail_lo = base * NS
    tail = [(tail_lo + m * CHUNK, min(CHUNK, total - tail_lo - m * CHUNK))
            for m in range(-(-(total - tail_lo) // CHUNK))]
    return base, main, tail


def _sc_scatter_body(p_hbm, src_hbm, dst_hbm, acc_out,
                     sidx_v, didx_v, rows_v, acc_sp, sem,
                     *, n_rows, n_pad, d, ept, do_gather):
    c = lax.axis_index("c")
    s = lax.axis_index("s")
    wid = c * NS + s
    n_chunks = ept // CHUNK

    def zero_row(i, carry):
        for k in range(d // L):
            rows_v[0, i, pl.ds(k * L, L)] = jnp.zeros((L,), jnp.float32)
        return carry
    lax.fori_loop(0, CHUNK, zero_row, 0)

    zbase, zmain, ztail = _pieces(n_pad)
    for off, sz in zmain:
        pltpu.sync_copy(rows_v.at[0].at[pl.ds(0, sz)],
                        acc_sp.at[pl.ds(pl.multiple_of(s * zbase + off, 8),
                                        sz)])

    @pl.when(s == NS - 1)
    def _():
        for off, sz in ztail:
            pltpu.sync_copy(rows_v.at[0].at[pl.ds(0, sz)],
                            acc_sp.at[pl.ds(off, sz)])

    plsc.subcore_barrier()

    nh = n_chunks // HALVES
    if do_gather:
        for h in range(HALVES):
            pltpu.sync_copy(src_hbm.at[wid].at[pl.ds(h * nh, nh)], sidx_v)
            pltpu.sync_copy(dst_hbm.at[wid].at[pl.ds(h * nh, nh)], didx_v)
            for b in range(NBUF):
                pltpu.async_copy(p_hbm.at[sidx_v.at[b]], rows_v.at[b], sem)

            def group(g, carry):
                j0 = g * NBUF
                for b in range(NBUF):
                    pltpu.make_async_copy(p_hbm.at[sidx_v.at[0]],
                                          rows_v.at[b], sem).wait()
                    pltpu.sync_copy(rows_v.at[b],
                                    acc_sp.at[didx_v.at[j0 + b]], add=True)

                    @pl.when(j0 + b + NBUF < nh)
                    def _():
                        pltpu.async_copy(p_hbm.at[sidx_v.at[j0 + b + NBUF]],
                                         rows_v.at[b], sem)
                return carry
            lax.fori_loop(0, nh // NBUF, group, 0)
    else:
        pltpu.sync_copy(p_hbm, rows_v.at[0])
        for h in range(HALVES):
            pltpu.sync_copy(dst_hbm.at[wid].at[pl.ds(h * nh, nh)], didx_v)

            def group(j, carry):
                pltpu.sync_copy(rows_v.at[0], acc_sp.at[didx_v.at[j]],
                                add=True)
                return carry
            lax.fori_loop(0, nh, group, 0)

    plsc.subcore_barrier()

    obase, omain, otail = _pieces(n_rows)

    def copy_piece(r0, sz):
        pltpu.sync_copy(acc_sp.at[pl.ds(r0, sz)], rows_v.at[0].at[pl.ds(0, sz)])
        pltpu.sync_copy(rows_v.at[0].at[pl.ds(0, sz)],
                        acc_out.at[c].at[pl.ds(r0, sz)])

    for off, sz in omain:
        copy_piece(pl.multiple_of(s * obase + off, 8), sz)

    @pl.when(s == NS - 1)
    def _():
        for off, sz in otail:
            copy_piece(off, sz)


@functools.lru_cache(maxsize=None)
def _make_sc_scatter(n_rows, n_pad, d, ept, do_gather=True):
    n_chunks = ept // CHUNK
    body = functools.partial(_sc_scatter_body, n_rows=n_rows, n_pad=n_pad,
                             d=d, ept=ept, do_gather=do_gather)
    mesh = plsc.VectorSubcoreMesh(core_axis_name="c", subcore_axis_name="s")
    return pl.kernel(
        body,
        out_type=jax.ShapeDtypeStruct((NC, n_rows, d), jnp.float32),
        mesh=mesh,
        scratch_types=(
            pltpu.VMEM((n_chunks // HALVES, CHUNK), jnp.int32),
            pltpu.VMEM((n_chunks // HALVES, CHUNK), jnp.int32),
            pltpu.VMEM((NBUF, CHUNK, d), jnp.float32),
            pltpu.VMEM_SHARED((n_pad, d), jnp.float32),
            pltpu.SemaphoreType.DMA,
        ),
        name=f"sage_scatter_g{int(do_gather)}")


def _mm_first_body(x_ref, wl_ref, wr_ref, bl_ref, p_ref, r_ref):
    x = x_ref[...]
    p_ref[...] = jnp.dot(x, wl_ref[...], preferred_element_type=jnp.float32)
    r_ref[...] = (jnp.dot(x, wr_ref[...], preferred_element_type=jnp.float32)
                  + bl_ref[...])


def _mid_body(a0_ref, a1_ref, c0_ref, c1_ref, r_ref, wl_ref, wr_ref, bl_ref,
              p_ref, rn_ref):
    cnt = c0_ref[:, :1] + c1_ref[:, :1]
    inv = 1.0 / jnp.maximum(cnt, 1.0)
    h = jnp.maximum((a0_ref[...] + a1_ref[...]) * inv + r_ref[...], 0.0)
    p_ref[...] = jnp.dot(h, wl_ref[...], preferred_element_type=jnp.float32)
    rn_ref[...] = (jnp.dot(h, wr_ref[...], preferred_element_type=jnp.float32)
                   + bl_ref[...])


def _final_body(a0_ref, a1_ref, c0_ref, c1_ref, r_ref, o_ref):
    cnt = c0_ref[:, :1] + c1_ref[:, :1]
    inv = 1.0 / jnp.maximum(cnt, 1.0)
    o_ref[...] = (a0_ref[...] + a1_ref[...]) * inv + r_ref[...]


def _row_spec(blk, d):
    return pl.BlockSpec((blk, d), lambda i: (i, 0))


def _full_spec(shape):
    return pl.BlockSpec(shape, lambda i: tuple(0 for _ in shape))


def _mm_first(x, wl, wr, bl, blk=1000):
    n, d = x.shape
    h = wl.shape[1]
    grid = (n // blk,)
    return pl.pallas_call(
        _mm_first_body,
        grid=grid,
        in_specs=[_row_spec(blk, d), _full_spec((d, h)), _full_spec((d, h)),
                  _full_spec((1, h))],
        out_specs=[_row_spec(blk, h), _row_spec(blk, h)],
        out_shape=[jax.ShapeDtypeStruct((n, h), jnp.float32),
                   jax.ShapeDtypeStruct((n, h), jnp.float32)],
    )(x, wl, wr, bl.reshape(1, h))


def _mm_mid(a0, a1, c0, c1, r, wl, wr, bl, blk=1000):
    n, d = a0.shape
    h = wl.shape[1]
    grid = (n // blk,)
    return pl.pallas_call(
        _mid_body,
        grid=grid,
        in_specs=[_row_spec(blk, d), _row_spec(blk, d), _row_spec(blk, d),
                  _row_spec(blk, d), _row_spec(blk, d), _full_spec((d, h)),
                  _full_spec((d, h)), _full_spec((1, h))],
        out_specs=[_row_spec(blk, h), _row_spec(blk, h)],
        out_shape=[jax.ShapeDtypeStruct((n, h), jnp.float32),
                   jax.ShapeDtypeStruct((n, h), jnp.float32)],
    )(a0, a1, c0, c1, r, wl, wr, bl.reshape(1, h))


def _mm_final(a0, a1, c0, c1, r, blk=1000):
    n, d = a0.shape
    grid = (n // blk,)
    return pl.pallas_call(
        _final_body,
        grid=grid,
        in_specs=[_row_spec(blk, d), _row_spec(blk, d), _row_spec(blk, d),
                  _row_spec(blk, d), _row_spec(blk, d)],
        out_specs=_row_spec(blk, d),
        out_shape=jax.ShapeDtypeStruct((n, d), jnp.float32),
    )(a0, a1, c0, c1, r)


def kernel(x, edge_index, Wl1, bl1, Wr1, Wl2, bl2, Wr2, Wl3, bl3, Wr3,
           Wl4, bl4, Wr4):
    n, d = x.shape
    e = edge_index.shape[1]
    src = edge_index[0].astype(jnp.int32)
    dst = edge_index[1].astype(jnp.int32)

    ept = -(-e // (NW * HALVES * NBUF * CHUNK)) * HALVES * NBUF * CHUNK
    epad = ept * NW
    per_tile_words = 2 * (ept // HALVES) + NBUF * CHUNK * d
    cap_rows = (2095104 - NS * per_tile_words) // d // 8 * 8
    n_pad = max(-(-(n + 1) // 8) * 8, min(cap_rows, n + 1024))
    if epad != e:
        src = jnp.pad(src, (0, epad - e))
        pad_dst = n + (jnp.arange(epad - e, dtype=jnp.int32) % (n_pad - n))
        dst = jnp.concatenate([dst, pad_dst])
    n_chunks = ept // CHUNK
    src = src.reshape(NW, n_chunks, CHUNK)
    dst = dst.reshape(NW, n_chunks, CHUNK)

    sc_scatter = _make_sc_scatter(n, n_pad, d, ept)
    sc_counts = _make_sc_scatter(n, n_pad, d, ept, do_gather=False)

    cnts = sc_counts(jnp.ones((CHUNK, d), jnp.float32), src, dst)
    c0, c1 = cnts[0], cnts[1]

    p, r = _mm_first(x, Wl1, Wr1, bl1)
    acc = sc_scatter(p, src, dst)
    p, r = _mm_mid(acc[0], acc[1], c0, c1, r, Wl2, Wr2, bl2)
    acc = sc_scatter(p, src, dst)
    p, r = _mm_mid(acc[0], acc[1], c0, c1, r, Wl3, Wr3, bl3)
    acc = sc_scatter(p, src, dst)
    p, r = _mm_mid(acc[0], acc[1], c0, c1, r, Wl4, Wr4, bl4)
    acc = sc_scatter(p, src, dst)
    return _mm_final(acc[0], acc[1], c0, c1, r)


# speedup vs baseline: 3.2776x; 3.2776x over previous
import functools

import jax
import jax.numpy as jnp
from jax import lax
from jax.experimental import pallas as pl
from jax.experimental.pallas import tpu as pltpu
from jax.experimental.pallas import tpu_sc as plsc

NC = 2
NS = 16
L = 16
NW = NC * NS
CHUNK = 128
NBUF = 2
HALVES = 2


def _pieces(total):
    base = (total // NS) // 8 * 8
    main = [(m * CHUNK, min(CHUNK, base - m * CHUNK))
            for m in range(-(-base // CHUNK))]
    tail_lo = base * NS
    tail = [(tail_lo + m * CHUNK, min(CHUNK, total - tail_lo - m * CHUNK))
            for m in range(-(-(total - tail_lo) // CHUNK))]
    return base, main, tail


def _sc_scatter_body(p_hbm, src_hbm, dst_hbm, acc_out,
                     sidx_v, didx_v, rows_v, acc_sp, sem,
                     *, n_rows, n_pad, d, ept, do_gather):
    c = lax.axis_index("c")
    s = lax.axis_index("s")
    wid = c * NS + s
    n_chunks = ept // CHUNK

    def zero_row(i, carry):
        for k in range(d // L):
            rows_v[0, i, pl.ds(k * L, L)] = jnp.zeros((L,), jnp.float32)
        return carry
    lax.fori_loop(0, CHUNK, zero_row, 0)

    zbase, zmain, ztail = _pieces(n_pad)
    for off, sz in zmain:
        pltpu.sync_copy(rows_v.at[0].at[pl.ds(0, sz)],
                        acc_sp.at[pl.ds(pl.multiple_of(s * zbase + off, 8),
                                        sz)])

    @pl.when(s == NS - 1)
    def _():
        for off, sz in ztail:
            pltpu.sync_copy(rows_v.at[0].at[pl.ds(0, sz)],
                            acc_sp.at[pl.ds(off, sz)])

    plsc.subcore_barrier()

    nh = n_chunks // HALVES
    if do_gather:
        for h in range(HALVES):
            pltpu.sync_copy(src_hbm.at[wid].at[pl.ds(h * nh, nh)], sidx_v)
            pltpu.sync_copy(dst_hbm.at[wid].at[pl.ds(h * nh, nh)], didx_v)
            for b in range(NBUF):
                pltpu.async_copy(p_hbm.at[sidx_v.at[b]], rows_v.at[b], sem)

            def group(g, carry):
                j0 = g * NBUF
                for b in range(NBUF):
                    pltpu.make_async_copy(p_hbm.at[sidx_v.at[0]],
                                          rows_v.at[b], sem).wait()
                    pltpu.sync_copy(rows_v.at[b],
                                    acc_sp.at[didx_v.at[j0 + b]], add=True)

                    @pl.when(j0 + b + NBUF < nh)
                    def _():
                        pltpu.async_copy(p_hbm.at[sidx_v.at[j0 + b + NBUF]],
                                         rows_v.at[b], sem)
                return carry
            lax.fori_loop(0, nh // NBUF, group, 0)
    else:
        pltpu.sync_copy(p_hbm, rows_v.at[0])
        for h in range(HALVES):
            pltpu.sync_copy(dst_hbm.at[wid].at[pl.ds(h * nh, nh)], didx_v)

            def group(j, carry):
                pltpu.sync_copy(rows_v.at[0], acc_sp.at[didx_v.at[j]],
                                add=True)
                return carry
            lax.fori_loop(0, nh, group, 0)

    plsc.subcore_barrier()

    obase, omain, otail = _pieces(n_rows)

    def copy_piece(r0, sz):
        pltpu.sync_copy(acc_sp.at[pl.ds(r0, sz)], rows_v.at[0].at[pl.ds(0, sz)])
        pltpu.sync_copy(rows_v.at[0].at[pl.ds(0, sz)],
                        acc_out.at[c].at[pl.ds(r0, sz)])

    for off, sz in omain:
        copy_piece(pl.multiple_of(s * obase + off, 8), sz)

    @pl.when(s == NS - 1)
    def _():
        for off, sz in otail:
            copy_piece(off, sz)


@functools.lru_cache(maxsize=None)
def _make_sc_scatter(n_rows, n_pad, d, ept, do_gather=True):
    n_chunks = ept // CHUNK
    body = functools.partial(_sc_scatter_body, n_rows=n_rows, n_pad=n_pad,
                             d=d, ept=ept, do_gather=do_gather)
    mesh = plsc.VectorSubcoreMesh(core_axis_name="c", subcore_axis_name="s")
    return pl.kernel(
        body,
        out_type=jax.ShapeDtypeStruct((NC, n_rows, d), jnp.float32),
        mesh=mesh,
        scratch_types=(
            pltpu.VMEM((n_chunks // HALVES, CHUNK), jnp.int32),
            pltpu.VMEM((n_chunks // HALVES, CHUNK), jnp.int32),
            pltpu.VMEM((NBUF, CHUNK, d), jnp.float32),
            pltpu.VMEM_SHARED((n_pad, d), jnp.float32),
            pltpu.SemaphoreType.DMA,
        ),
        name=f"sage_scatter_g{int(do_gather)}")


def _mm_first_body(x_ref, wl_ref, wr_ref, bl_ref, p_ref, r_ref):
    x = x_ref[...]
    p_ref[...] = jnp.dot(x, wl_ref[...], preferred_element_type=jnp.float32)
    r_ref[...] = (jnp.dot(x, wr_ref[...], preferred_element_type=jnp.float32)
                  + bl_ref[...])


def _mid_body(a0_ref, a1_ref, c0_ref, c1_ref, r_ref, wl_ref, wr_ref, bl_ref,
              p_ref, rn_ref):
    cnt = c0_ref[:, :1] + c1_ref[:, :1]
    inv = 1.0 / jnp.maximum(cnt, 1.0)
    h = jnp.maximum((a0_ref[...] + a1_ref[...]) * inv + r_ref[...], 0.0)
    p_ref[...] = jnp.dot(h, wl_ref[...], preferred_element_type=jnp.float32)
    rn_ref[...] = (jnp.dot(h, wr_ref[...], preferred_element_type=jnp.float32)
                   + bl_ref[...])


def _final_body(a0_ref, a1_ref, c0_ref, c1_ref, r_ref, o_ref):
    cnt = c0_ref[:, :1] + c1_ref[:, :1]
    inv = 1.0 / jnp.maximum(cnt, 1.0)
    o_ref[...] = (a0_ref[...] + a1_ref[...]) * inv + r_ref[...]


def _row_spec(blk, d):
    return pl.BlockSpec((blk, d), lambda i: (i, 0))


def _full_spec(shape):
    return pl.BlockSpec(shape, lambda i: tuple(0 for _ in shape))


def _mm_first(x, wl, wr, bl, blk=1000):
    n, d = x.shape
    h = wl.shape[1]
    grid = (n // blk,)
    return pl.pallas_call(
        _mm_first_body,
        grid=grid,
        in_specs=[_row_spec(blk, d), _full_spec((d, h)), _full_spec((d, h)),
                  _full_spec((1, h))],
        out_specs=[_row_spec(blk, h), _row_spec(blk, h)],
        out_shape=[jax.ShapeDtypeStruct((n, h), jnp.float32),
                   jax.ShapeDtypeStruct((n, h), jnp.float32)],
    )(x, wl, wr, bl.reshape(1, h))


def _mm_mid(a0, a1, c0, c1, r, wl, wr, bl, blk=1000):
    n, d = a0.shape
    h = wl.shape[1]
    grid = (n // blk,)
    return pl.pallas_call(
        _mid_body,
        grid=grid,
        in_specs=[_row_spec(blk, d), _row_spec(blk, d), _row_spec(blk, d),
                  _row_spec(blk, d), _row_spec(blk, d), _full_spec((d, h)),
                  _full_spec((d, h)), _full_spec((1, h))],
        out_specs=[_row_spec(blk, h), _row_spec(blk, h)],
        out_shape=[jax.ShapeDtypeStruct((n, h), jnp.float32),
                   jax.ShapeDtypeStruct((n, h), jnp.float32)],
    )(a0, a1, c0, c1, r, wl, wr, bl.reshape(1, h))


def _mm_final(a0, a1, c0, c1, r, blk=1000):
    n, d = a0.shape
    grid = (n // blk,)
    return pl.pallas_call(
        _final_body,
        grid=grid,
        in_specs=[_row_spec(blk, d), _row_spec(blk, d), _row_spec(blk, d),
                  _row_spec(blk, d), _row_spec(blk, d)],
        out_specs=_row_spec(blk, d),
        out_shape=jax.ShapeDtypeStruct((n, d), jnp.float32),
    )(a0, a1, c0, c1, r)


def kernel(x, edge_index, Wl1, bl1, Wr1, Wl2, bl2, Wr2, Wl3, bl3, Wr3,
           Wl4, bl4, Wr4):
    n, d = x.shape
    e = edge_index.shape[1]
    src = edge_index[0].astype(jnp.int32)
    dst = edge_index[1].astype(jnp.int32)

    ept = -(-e // (NW * HALVES * NBUF * CHUNK)) * HALVES * NBUF * CHUNK
    epad = ept * NW
    per_tile_words = 2 * (ept // HALVES) + NBUF * CHUNK * d
    cap_rows = (2095104 - NS * per_tile_words) // d // 8 * 8
    n_pad = max(-(-(n + 1) // 8) * 8, min(cap_rows, n + 1024))
    if epad != e:
        pad_idx = jnp.arange(epad - e, dtype=jnp.int32)
        src = jnp.concatenate([src, pad_idx % n])
        dst = jnp.concatenate([dst, n + pad_idx % (n_pad - n)])
    n_chunks = ept // CHUNK
    src = src.reshape(NW, n_chunks, CHUNK)
    dst = dst.reshape(NW, n_chunks, CHUNK)

    sc_scatter = _make_sc_scatter(n, n_pad, d, ept)
    sc_counts = _make_sc_scatter(n, n_pad, d, ept, do_gather=False)

    cnts = sc_counts(jnp.ones((CHUNK, d), jnp.float32), src, dst)
    c0, c1 = cnts[0], cnts[1]

    p, r = _mm_first(x, Wl1, Wr1, bl1)
    acc = sc_scatter(p, src, dst)
    p, r = _mm_mid(acc[0], acc[1], c0, c1, r, Wl2, Wr2, bl2)
    acc = sc_scatter(p, src, dst)
    p, r = _mm_mid(acc[0], acc[1], c0, c1, r, Wl3, Wr3, bl3)
    acc = sc_scatter(p, src, dst)
    p, r = _mm_mid(acc[0], acc[1], c0, c1, r, Wl4, Wr4, bl4)
    acc = sc_scatter(p, src, dst)
    return _mm_final(acc[0], acc[1], c0, c1, r)


# speedup vs baseline: 3.4335x; 1.0476x over previous
import functools

import jax
import jax.numpy as jnp
from jax import lax
from jax.experimental import pallas as pl
from jax.experimental.pallas import tpu as pltpu
from jax.experimental.pallas import tpu_sc as plsc

NC = 2
NS = 16
L = 16
NW = NC * NS
CHUNK = 128
NBUF = 2
HALVES = 2


def _pieces(total):
    base = (total // NS) // 8 * 8
    main = [(m * CHUNK, min(CHUNK, base - m * CHUNK))
            for m in range(-(-base // CHUNK))]
    tail_lo = base * NS
    tail = [(tail_lo + m * CHUNK, min(CHUNK, total - tail_lo - m * CHUNK))
            for m in range(-(-(total - tail_lo) // CHUNK))]
    return base, main, tail


def _sc_scatter_body(p_hbm, src_hbm, dst_hbm, acc_out,
                     sidx_v, didx_v, rows_v, acc_sp, sem,
                     *, n_rows, n_pad, d, ept, do_gather):
    c = lax.axis_index("c")
    s = lax.axis_index("s")
    wid = c * NS + s
    n_chunks = ept // CHUNK

    def zero_row(i, carry):
        for k in range(d // L):
            rows_v[0, i, pl.ds(k * L, L)] = jnp.zeros((L,), jnp.float32)
        return carry
    lax.fori_loop(0, CHUNK, zero_row, 0)

    zbase, zmain, ztail = _pieces(n_pad)
    for off, sz in zmain:
        pltpu.sync_copy(rows_v.at[0].at[pl.ds(0, sz)],
                        acc_sp.at[pl.ds(pl.multiple_of(s * zbase + off, 8),
                                        sz)])

    @pl.when(s == NS - 1)
    def _():
        for off, sz in ztail:
            pltpu.sync_copy(rows_v.at[0].at[pl.ds(0, sz)],
                            acc_sp.at[pl.ds(off, sz)])

    plsc.subcore_barrier()

    nh = n_chunks // HALVES
    if do_gather:
        for h in range(HALVES):
            pltpu.sync_copy(src_hbm.at[wid].at[pl.ds(h * nh, nh)], sidx_v)
            pltpu.sync_copy(dst_hbm.at[wid].at[pl.ds(h * nh, nh)], didx_v)
            for b in range(NBUF):
                pltpu.async_copy(p_hbm.at[sidx_v.at[b]], rows_v.at[b], sem)

            def group(g, carry):
                j0 = g * NBUF
                for b in range(NBUF):
                    pltpu.make_async_copy(p_hbm.at[sidx_v.at[0]],
                                          rows_v.at[b], sem).wait()
                    pltpu.sync_copy(rows_v.at[b],
                                    acc_sp.at[didx_v.at[j0 + b]], add=True)

                    @pl.when(j0 + b + NBUF < nh)
                    def _():
                        pltpu.async_copy(p_hbm.at[sidx_v.at[j0 + b + NBUF]],
                                         rows_v.at[b], sem)
                return carry
            lax.fori_loop(0, nh // NBUF, group, 0)
    else:
        pltpu.sync_copy(p_hbm, rows_v.at[0])
        for h in range(HALVES):
            pltpu.sync_copy(dst_hbm.at[wid].at[pl.ds(h * nh, nh)], didx_v)

            def group(j, carry):
                pltpu.sync_copy(rows_v.at[0], acc_sp.at[didx_v.at[j]],
                                add=True)
                return carry
            lax.fori_loop(0, nh, group, 0)

    plsc.subcore_barrier()

    obase, omain, otail = _pieces(n_rows)

    def copy_piece(r0, sz):
        pltpu.sync_copy(acc_sp.at[pl.ds(r0, sz)], rows_v.at[0].at[pl.ds(0, sz)])
        pltpu.sync_copy(rows_v.at[0].at[pl.ds(0, sz)],
                        acc_out.at[c].at[pl.ds(r0, sz)])

    for off, sz in omain:
        copy_piece(pl.multiple_of(s * obase + off, 8), sz)

    @pl.when(s == NS - 1)
    def _():
        for off, sz in otail:
            copy_piece(off, sz)


@functools.lru_cache(maxsize=None)
def _make_sc_scatter(n_rows, n_pad, d, ept, do_gather=True):
    n_chunks = ept // CHUNK
    body = functools.partial(_sc_scatter_body, n_rows=n_rows, n_pad=n_pad,
                             d=d, ept=ept, do_gather=do_gather)
    mesh = plsc.VectorSubcoreMesh(core_axis_name="c", subcore_axis_name="s")
    return pl.kernel(
        body,
        out_type=jax.ShapeDtypeStruct((NC, n_rows, d), jnp.float32),
        mesh=mesh,
        scratch_types=(
            pltpu.VMEM((n_chunks // HALVES, CHUNK), jnp.int32),
            pltpu.VMEM((n_chunks // HALVES, CHUNK), jnp.int32),
            pltpu.VMEM((NBUF, CHUNK, d), jnp.float32),
            pltpu.VMEM_SHARED((n_pad, d), jnp.float32),
            pltpu.SemaphoreType.DMA,
        ),
        name=f"sage_scatter_g{int(do_gather)}")


def _mm_first_body(x_ref, wl_ref, wr_ref, bl_ref, p_ref, r_ref):
    x = x_ref[...]
    p_ref[...] = jnp.dot(x, wl_ref[...], preferred_element_type=jnp.float32)
    r_ref[...] = (jnp.dot(x, wr_ref[...], preferred_element_type=jnp.float32)
                  + bl_ref[...])


def _mid_body(a_ref, c_ref, r_ref, wl_ref, wr_ref, bl_ref,
              p_ref, rn_ref):
    cnt = c_ref[0, :, :1] + c_ref[1, :, :1]
    inv = 1.0 / jnp.maximum(cnt, 1.0)
    h = jnp.maximum((a_ref[0] + a_ref[1]) * inv + r_ref[...], 0.0)
    p_ref[...] = jnp.dot(h, wl_ref[...], preferred_element_type=jnp.float32)
    rn_ref[...] = (jnp.dot(h, wr_ref[...], preferred_element_type=jnp.float32)
                   + bl_ref[...])


def _final_body(a_ref, c_ref, r_ref, o_ref):
    cnt = c_ref[0, :, :1] + c_ref[1, :, :1]
    inv = 1.0 / jnp.maximum(cnt, 1.0)
    o_ref[...] = (a_ref[0] + a_ref[1]) * inv + r_ref[...]


def _row_spec(blk, d):
    return pl.BlockSpec((blk, d), lambda i: (i, 0))


def _full_spec(shape):
    return pl.BlockSpec(shape, lambda i: tuple(0 for _ in shape))


def _mm_first(x, wl, wr, bl, blk=1000):
    n, d = x.shape
    h = wl.shape[1]
    grid = (n // blk,)
    return pl.pallas_call(
        _mm_first_body,
        grid=grid,
        in_specs=[_row_spec(blk, d), _full_spec((d, h)), _full_spec((d, h)),
                  _full_spec((1, h))],
        out_specs=[_row_spec(blk, h), _row_spec(blk, h)],
        out_shape=[jax.ShapeDtypeStruct((n, h), jnp.float32),
                   jax.ShapeDtypeStruct((n, h), jnp.float32)],
    )(x, wl, wr, bl.reshape(1, h))


def _pair_spec(blk, d):
    return pl.BlockSpec((2, blk, d), lambda i: (0, i, 0))


def _mm_mid(acc, cnts, r, wl, wr, bl, blk=1000):
    _, n, d = acc.shape
    h = wl.shape[1]
    grid = (n // blk,)
    return pl.pallas_call(
        _mid_body,
        grid=grid,
        in_specs=[_pair_spec(blk, d), _pair_spec(blk, d), _row_spec(blk, d),
                  _full_spec((d, h)), _full_spec((d, h)), _full_spec((1, h))],
        out_specs=[_row_spec(blk, h), _row_spec(blk, h)],
        out_shape=[jax.ShapeDtypeStruct((n, h), jnp.float32),
                   jax.ShapeDtypeStruct((n, h), jnp.float32)],
    )(acc, cnts, r, wl, wr, bl.reshape(1, h))


def _mm_final(acc, cnts, r, blk=1000):
    _, n, d = acc.shape
    grid = (n // blk,)
    return pl.pallas_call(
        _final_body,
        grid=grid,
        in_specs=[_pair_spec(blk, d), _pair_spec(blk, d), _row_spec(blk, d)],
        out_specs=_row_spec(blk, d),
        out_shape=jax.ShapeDtypeStruct((n, d), jnp.float32),
    )(acc, cnts, r)


def kernel(x, edge_index, Wl1, bl1, Wr1, Wl2, bl2, Wr2, Wl3, bl3, Wr3,
           Wl4, bl4, Wr4):
    n, d = x.shape
    e = edge_index.shape[1]
    src = edge_index[0].astype(jnp.int32)
    dst = edge_index[1].astype(jnp.int32)

    ept = -(-e // (NW * HALVES * NBUF * CHUNK)) * HALVES * NBUF * CHUNK
    epad = ept * NW
    per_tile_words = 2 * (ept // HALVES) + NBUF * CHUNK * d
    cap_rows = (2095104 - NS * per_tile_words) // d // 8 * 8
    n_pad = max(-(-(n + 1) // 8) * 8, min(cap_rows, n + 1024))
    if epad != e:
        pad_idx = jnp.arange(epad - e, dtype=jnp.int32)
        src = jnp.concatenate([src, pad_idx % n])
        dst = jnp.concatenate([dst, n + pad_idx % (n_pad - n)])
    n_chunks = ept // CHUNK
    src = src.reshape(NW, n_chunks, CHUNK)
    dst = dst.reshape(NW, n_chunks, CHUNK)

    sc_scatter = _make_sc_scatter(n, n_pad, d, ept)
    sc_counts = _make_sc_scatter(n, n_pad, d, ept, do_gather=False)

    cnts = sc_counts(jnp.ones((CHUNK, d), jnp.float32), src, dst)

    p, r = _mm_first(x, Wl1, Wr1, bl1)
    acc = sc_scatter(p, src, dst)
    p, r = _mm_mid(acc, cnts, r, Wl2, Wr2, bl2)
    acc = sc_scatter(p, src, dst)
    p, r = _mm_mid(acc, cnts, r, Wl3, Wr3, bl3)
    acc = sc_scatter(p, src, dst)
    p, r = _mm_mid(acc, cnts, r, Wl4, Wr4, bl4)
    acc = sc_scatter(p, src, dst)
    return _mm_final(acc, cnts, r)


# speedup vs baseline: 3.5081x; 1.0217x over previous
import functools

import jax
import jax.numpy as jnp
from jax import lax
from jax.experimental import pallas as pl
from jax.experimental.pallas import tpu as pltpu
from jax.experimental.pallas import tpu_sc as plsc

NC = 2
NS = 16
L = 16
NW = NC * NS
CHUNK = 128
NBUF = 2
HALVES = 2


def _pieces(total):
    base = (total // NS) // 8 * 8
    main = [(m * CHUNK, min(CHUNK, base - m * CHUNK))
            for m in range(-(-base // CHUNK))]
    tail_lo = base * NS
    tail = [(tail_lo + m * CHUNK, min(CHUNK, total - tail_lo - m * CHUNK))
            for m in range(-(-(total - tail_lo) // CHUNK))]
    return base, main, tail


def _sc_scatter_body(p_hbm, src_hbm, dst_hbm, acc_out,
                     sidx_v, didx_v, rows_v, acc_sp, sem,
                     *, n_rows, n_pad, d, ept, do_gather):
    c = lax.axis_index("c")
    s = lax.axis_index("s")
    wid = c * NS + s
    n_chunks = ept // CHUNK

    def zero_row(i, carry):
        for k in range(d // L):
            rows_v[0, i, pl.ds(k * L, L)] = jnp.zeros((L,), jnp.float32)
        return carry
    lax.fori_loop(0, CHUNK, zero_row, 0)

    zbase, zmain, ztail = _pieces(n_pad)
    for off, sz in zmain:
        pltpu.sync_copy(rows_v.at[0].at[pl.ds(0, sz)],
                        acc_sp.at[pl.ds(pl.multiple_of(s * zbase + off, 8),
                                        sz)])

    @pl.when(s == NS - 1)
    def _():
        for off, sz in ztail:
            pltpu.sync_copy(rows_v.at[0].at[pl.ds(0, sz)],
                            acc_sp.at[pl.ds(off, sz)])

    plsc.subcore_barrier()

    nh = n_chunks // HALVES
    if do_gather:
        for h in range(HALVES):
            pltpu.sync_copy(src_hbm.at[wid].at[pl.ds(h * nh, nh)], sidx_v)
            pltpu.sync_copy(dst_hbm.at[wid].at[pl.ds(h * nh, nh)], didx_v)
            for b in range(NBUF):
                pltpu.async_copy(p_hbm.at[sidx_v.at[b]], rows_v.at[b], sem)

            def group(g, carry):
                j0 = g * NBUF
                for b in range(NBUF):
                    pltpu.make_async_copy(p_hbm.at[sidx_v.at[0]],
                                          rows_v.at[b], sem).wait()
                    pltpu.sync_copy(rows_v.at[b],
                                    acc_sp.at[didx_v.at[j0 + b]], add=True)

                    @pl.when(j0 + b + NBUF < nh)
                    def _():
                        pltpu.async_copy(p_hbm.at[sidx_v.at[j0 + b + NBUF]],
                                         rows_v.at[b], sem)
                return carry
            lax.fori_loop(0, nh // NBUF, group, 0)
    else:
        pltpu.sync_copy(p_hbm, rows_v.at[0])
        for h in range(HALVES):
            pltpu.sync_copy(dst_hbm.at[wid].at[pl.ds(h * nh, nh)], didx_v)

            def group(j, carry):
                pltpu.sync_copy(rows_v.at[0], acc_sp.at[didx_v.at[j]],
                                add=True)
                return carry
            lax.fori_loop(0, nh, group, 0)

    plsc.subcore_barrier()

    obase, omain, otail = _pieces(n_rows)

    def copy_piece(r0, sz):
        pltpu.sync_copy(acc_sp.at[pl.ds(r0, sz)],
                        acc_out.at[c].at[pl.ds(r0, sz)])

    for off, sz in omain:
        copy_piece(pl.multiple_of(s * obase + off, 8), sz)

    @pl.when(s == NS - 1)
    def _():
        for off, sz in otail:
            copy_piece(off, sz)


@functools.lru_cache(maxsize=None)
def _make_sc_scatter(n_rows, n_pad, d, ept, do_gather=True):
    n_chunks = ept // CHUNK
    body = functools.partial(_sc_scatter_body, n_rows=n_rows, n_pad=n_pad,
                             d=d, ept=ept, do_gather=do_gather)
    mesh = plsc.VectorSubcoreMesh(core_axis_name="c", subcore_axis_name="s")
    return pl.kernel(
        body,
        out_type=jax.ShapeDtypeStruct((NC, n_rows, d), jnp.float32),
        mesh=mesh,
        scratch_types=(
            pltpu.VMEM((n_chunks // HALVES, CHUNK), jnp.int32),
            pltpu.VMEM((n_chunks // HALVES, CHUNK), jnp.int32),
            pltpu.VMEM((NBUF, CHUNK, d), jnp.float32),
            pltpu.VMEM_SHARED((n_pad, d), jnp.float32),
            pltpu.SemaphoreType.DMA,
        ),
        name=f"sage_scatter_g{int(do_gather)}")


def _mm_first_body(x_ref, wl_ref, wr_ref, bl_ref, p_ref, r_ref):
    x = x_ref[...]
    p_ref[...] = jnp.dot(x, wl_ref[...], preferred_element_type=jnp.float32)
    r_ref[...] = (jnp.dot(x, wr_ref[...], preferred_element_type=jnp.float32)
                  + bl_ref[...])


def _mid_body(a_ref, c_ref, r_ref, wl_ref, wr_ref, bl_ref,
              p_ref, rn_ref):
    cnt = c_ref[0, :, :1] + c_ref[1, :, :1]
    inv = 1.0 / jnp.maximum(cnt, 1.0)
    h = jnp.maximum((a_ref[0] + a_ref[1]) * inv + r_ref[...], 0.0)
    p_ref[...] = jnp.dot(h, wl_ref[...], preferred_element_type=jnp.float32)
    rn_ref[...] = (jnp.dot(h, wr_ref[...], preferred_element_type=jnp.float32)
                   + bl_ref[...])


def _final_body(a_ref, c_ref, r_ref, o_ref):
    cnt = c_ref[0, :, :1] + c_ref[1, :, :1]
    inv = 1.0 / jnp.maximum(cnt, 1.0)
    o_ref[...] = (a_ref[0] + a_ref[1]) * inv + r_ref[...]


def _row_spec(blk, d):
    return pl.BlockSpec((blk, d), lambda i: (i, 0))


def _full_spec(shape):
    return pl.BlockSpec(shape, lambda i: tuple(0 for _ in shape))


def _mm_first(x, wl, wr, bl, blk=2000):
    n, d = x.shape
    h = wl.shape[1]
    grid = (n // blk,)
    return pl.pallas_call(
        _mm_first_body,
        grid=grid,
        in_specs=[_row_spec(blk, d), _full_spec((d, h)), _full_spec((d, h)),
                  _full_spec((1, h))],
        out_specs=[_row_spec(blk, h), _row_spec(blk, h)],
        out_shape=[jax.ShapeDtypeStruct((n, h), jnp.float32),
                   jax.ShapeDtypeStruct((n, h), jnp.float32)],
    )(x, wl, wr, bl.reshape(1, h))


def _pair_spec(blk, d):
    return pl.BlockSpec((2, blk, d), lambda i: (0, i, 0))


def _mm_mid(acc, cnts, r, wl, wr, bl, blk=2000):
    _, n, d = acc.shape
    h = wl.shape[1]
    grid = (n // blk,)
    return pl.pallas_call(
        _mid_body,
        grid=grid,
        in_specs=[_pair_spec(blk, d), _pair_spec(blk, d), _row_spec(blk, d),
                  _full_spec((d, h)), _full_spec((d, h)), _full_spec((1, h))],
        out_specs=[_row_spec(blk, h), _row_spec(blk, h)],
        out_shape=[jax.ShapeDtypeStruct((n, h), jnp.float32),
                   jax.ShapeDtypeStruct((n, h), jnp.float32)],
    )(acc, cnts, r, wl, wr, bl.reshape(1, h))


def _mm_final(acc, cnts, r, blk=2000):
    _, n, d = acc.shape
    grid = (n // blk,)
    return pl.pallas_call(
        _final_body,
        grid=grid,
        in_specs=[_pair_spec(blk, d), _pair_spec(blk, d), _row_spec(blk, d)],
        out_specs=_row_spec(blk, d),
        out_shape=jax.ShapeDtypeStruct((n, d), jnp.float32),
    )(acc, cnts, r)


def kernel(x, edge_index, Wl1, bl1, Wr1, Wl2, bl2, Wr2, Wl3, bl3, Wr3,
           Wl4, bl4, Wr4):
    n, d = x.shape
    e = edge_index.shape[1]
    src = edge_index[0].astype(jnp.int32)
    dst = edge_index[1].astype(jnp.int32)

    ept = -(-e // (NW * HALVES * NBUF * CHUNK)) * HALVES * NBUF * CHUNK
    epad = ept * NW
    per_tile_words = 2 * (ept // HALVES) + NBUF * CHUNK * d
    cap_rows = (2095104 - NS * per_tile_words) // d // 8 * 8
    n_pad = max(-(-(n + 1) // 8) * 8, min(cap_rows, n + 1024))
    if epad != e:
        pad_idx = jnp.arange(epad - e, dtype=jnp.int32)
        src = jnp.concatenate([src, pad_idx % n])
        dst = jnp.concatenate([dst, n + pad_idx % (n_pad - n)])
    n_chunks = ept // CHUNK
    src = src.reshape(NW, n_chunks, CHUNK)
    dst = dst.reshape(NW, n_chunks, CHUNK)

    sc_scatter = _make_sc_scatter(n, n_pad, d, ept)
    sc_counts = _make_sc_scatter(n, n_pad, d, ept, do_gather=False)

    cnts = sc_counts(jnp.ones((CHUNK, d), jnp.float32), src, dst)

    p, r = _mm_first(x, Wl1, Wr1, bl1)
    acc = sc_scatter(p, src, dst)
    p, r = _mm_mid(acc, cnts, r, Wl2, Wr2, bl2)
    acc = sc_scatter(p, src, dst)
    p, r = _mm_mid(acc, cnts, r, Wl3, Wr3, bl3)
    acc = sc_scatter(p, src, dst)
    p, r = _mm_mid(acc, cnts, r, Wl4, Wr4, bl4)
    acc = sc_scatter(p, src, dst)
    return _mm_final(acc, cnts, r)
